# baseline (speedup 1.0000x reference)
import functools

import jax
import jax.numpy as jnp
from jax import lax
from jax.experimental import pallas as pl
from jax.experimental.pallas import tpu as pltpu
from jax.experimental.pallas import tpu_sc as plsc

V = 2000
V2 = 2048
NUM_ITEMS = V - 2
PAD_ID = NUM_ITEMS
HID = 32
P = 16
NPAD = 1024
RPAD = 40


def _pad_body(src_ref, dst_ref, osrc_ref, odst_ref):
    osrc_ref[:, 0:V] = src_ref[...]
    odst_ref[:, 0:V] = dst_ref[...]


def _pad_tables(st_src, st_dst):
    nblk = V // RPAD
    return pl.pallas_call(
        _pad_body,
        grid=(nblk,),
        in_specs=[
            pl.BlockSpec((RPAD, V), lambda i: (i, 0)),
            pl.BlockSpec((RPAD, V), lambda i: (i, 0)),
        ],
        out_specs=[
            pl.BlockSpec((RPAD, V2), lambda i: (i, 0)),
            pl.BlockSpec((RPAD, V2), lambda i: (i, 0)),
        ],
        out_shape=[jax.ShapeDtypeStruct((V, V2), jnp.float32),
                   jax.ShapeDtypeStruct((V, V2), jnp.float32)],
    )(st_src, st_dst)


def _make_sc_gather(n):
    info = plsc.get_sparse_core_info()
    nc, ns = info.num_cores, info.num_subcores
    nw = nc * ns
    b_per_w = NPAD // nw
    mesh = plsc.VectorSubcoreMesh(core_axis_name="c", subcore_axis_name="s")

    @functools.partial(
        pl.kernel, mesh=mesh,
        out_type=(pltpu.HBM((NPAD, V2), jnp.float32),
                  pltpu.HBM((NPAD, V2), jnp.float32)),
        scratch_types=[
            pltpu.VMEM((b_per_w,), jnp.int32),
            pltpu.VMEM((b_per_w, V2), jnp.float32),
            pltpu.SemaphoreType.DMA,
        ],
    )
    def sc_gather(src_hbm, dst_hbm, sidx_hbm, didx_hbm,
                  out_src, out_dst, idx_v, rows, sem):
        wid = lax.axis_index("s") * nc + lax.axis_index("c")
        base = wid * b_per_w
        nactive = -(-n // b_per_w)

        @pl.when(wid < nactive)
        def _():
            for idx_hbm, table, out in ((sidx_hbm, src_hbm, out_src),
                                        (didx_hbm, dst_hbm, out_dst)):
                pltpu.sync_copy(idx_hbm.at[pl.ds(base, b_per_w)], idx_v)
                pltpu.async_copy(table.at[idx_v], rows, sem).wait()
                pltpu.sync_copy(rows, out.at[pl.ds(base, b_per_w)])

    return sc_gather


_GK = 0.7978845608028654
_GKA = _GK * 0.044715


def _tc_body(src_ref, dst_ref, pop_ref, dmat_ref,
             w1t_ref, w2h_ref, b2_ref, lab_ref, out_ref):
    pop = pop_ref[...]
    inv = jnp.where(pop == 0.0, 0.0,
                    1.0 / jnp.where(pop == 0.0, 1.0, pop))
    a = src_ref[...] * inv
    b = dst_ref[...] * inv

    acc = jnp.zeros((P, V2), jnp.float32)
    for h in range(HID):
        u = w1t_ref[h:h + 1, 0:1]
        w = w1t_ref[h:h + 1, 1:2]
        x = a * u + b * w + dmat_ref[h:h + 1, :]
        s = x * x
        th = jnp.tanh(x * (s * _GKA + _GK))
        v = x * w2h_ref[h:h + 1, 0:1]
        acc = acc + v + v * th
    logits = acc + b2_ref[0:1, 0:1]

    col = lax.broadcasted_iota(jnp.int32, (1, V2), 1)
    valid = col < NUM_ITEMS
    neg = jnp.where(valid, logits, -jnp.inf)
    m = jnp.max(neg, axis=1, keepdims=True)
    s = jnp.sum(jnp.where(valid, jnp.exp(logits - m), 0.0),
                axis=1, keepdims=True)
    lab = lab_ref[0]
    pick = jnp.sum(jnp.where(col == lab, logits, 0.0),
                   axis=1, keepdims=True)
    loss = jnp.log(s) + m - pick
    out_ref[...] = loss.reshape(1, P, 1)


def _tc_loss(src_g, dst_g, pop2, popn, W1, b1, W2, b2, lab, n):
    nblk = n // P
    dmat = jnp.pad(W1[2][:, None] * popn + b1[:, None],
                   ((0, 0), (0, V2 - V)))
    return pl.pallas_call(
        _tc_body,
        grid=(nblk,),
        in_specs=[
            pl.BlockSpec((P, V2), lambda i: (i, 0)),
            pl.BlockSpec((P, V2), lambda i: (i, 0)),
            pl.BlockSpec((1, V2), lambda i: (0, 0)),
            pl.BlockSpec((HID, V2), lambda i: (0, 0)),
            pl.BlockSpec((HID, 3), lambda i: (0, 0)),
            pl.BlockSpec((HID, 1), lambda i: (0, 0)),
            pl.BlockSpec((1, 1), lambda i: (0, 0)),
            pl.BlockSpec((1, P, 1), lambda i: (i, 0, 0)),
        ],
        out_specs=pl.BlockSpec((1, P, 1), lambda i: (i, 0, 0)),
        out_shape=jax.ShapeDtypeStruct((nblk, P, 1), jnp.float32),
    )(src_g, dst_g, pop2, dmat,
      W1.T, (0.5 * W2), b2.reshape(1, 1),
      lab.reshape(nblk, P, 1)).reshape(n)


@jax.jit
def kernel(st_src, st_dst, pop_biases, pop_biases_norm, W1, b1, W2, b2,
           masked_sequences, labels, positions):
    B, L = masked_sequences.shape
    n = B * L

    seqs = jnp.maximum(masked_sequences, 0)
    pad_col = jnp.full((B, 1), PAD_ID, dtype=seqs.dtype)
    src_idx = jnp.concatenate([pad_col, seqs[:, :-1]], axis=1).reshape(n)
    dst_idx = jnp.concatenate([seqs[:, 1:], pad_col], axis=1).reshape(n)
    zpad = jnp.zeros((NPAD - n,), jnp.int32)
    src_idx = jnp.concatenate([src_idx, zpad])
    dst_idx = jnp.concatenate([dst_idx, zpad])
    lab = jnp.maximum(labels, 0).reshape(n)

    src2, dst2 = _pad_tables(st_src, st_dst)
    pop2 = jnp.pad(pop_biases, ((0, 0), (0, V2 - V)))

    src_g, dst_g = _make_sc_gather(n)(src2, dst2, src_idx, dst_idx)

    return _tc_loss(src_g, dst_g, pop2, pop_biases_norm.reshape(1, V),
                    W1, b1, W2, b2, lab, n)


# speedup vs baseline: 1.1004x; 1.1004x over previous
import functools

import jax
import jax.numpy as jnp
from jax import lax
from jax.experimental import pallas as pl
from jax.experimental.pallas import tpu as pltpu
from jax.experimental.pallas import tpu_sc as plsc

V = 2000
V2 = 2048
NUM_ITEMS = V - 2
PAD_ID = NUM_ITEMS
HID = 32
P = 16
NPAD = 1024
RPAD = 40


def _pad_body(src_ref, dst_ref, osrc_ref, odst_ref):
    osrc_ref[:, 0:V] = src_ref[...]
    odst_ref[:, 0:V] = dst_ref[...]


def _pad_tables(st_src, st_dst):
    nblk = V // RPAD
    return pl.pallas_call(
        _pad_body,
        grid=(nblk,),
        in_specs=[
            pl.BlockSpec((RPAD, V), lambda i: (i, 0)),
            pl.BlockSpec((RPAD, V), lambda i: (i, 0)),
        ],
        out_specs=[
            pl.BlockSpec((RPAD, V2), lambda i: (i, 0)),
            pl.BlockSpec((RPAD, V2), lambda i: (i, 0)),
        ],
        out_shape=[jax.ShapeDtypeStruct((V, V2), jnp.float32),
                   jax.ShapeDtypeStruct((V, V2), jnp.float32)],
    )(st_src, st_dst)


def _make_sc_gather(n):
    info = plsc.get_sparse_core_info()
    nc, ns = info.num_cores, info.num_subcores
    nw = nc * ns
    b_per_w = NPAD // nw
    mesh = plsc.VectorSubcoreMesh(core_axis_name="c", subcore_axis_name="s")

    @functools.partial(
        pl.kernel, mesh=mesh,
        out_type=pltpu.HBM((NPAD, V2), jnp.float32),
        scratch_types=[
            pltpu.VMEM((b_per_w,), jnp.int32),
            pltpu.VMEM((b_per_w, V2), jnp.float32),
            pltpu.SemaphoreType.DMA,
        ],
    )
    def sc_gather(table_hbm, idx_hbm, out, idx_v, rows, sem):
        wid = lax.axis_index("s") * nc + lax.axis_index("c")
        base = wid * b_per_w
        nactive = -(-n // b_per_w)

        @pl.when(wid < nactive)
        def _():
            pltpu.sync_copy(idx_hbm.at[pl.ds(base, b_per_w)], idx_v)
            pltpu.async_copy(table_hbm.at[idx_v], rows, sem).wait()
            pltpu.sync_copy(rows, out.at[pl.ds(base, b_per_w)])

    return sc_gather


_GK = 0.7978845608028654
_GKA = _GK * 0.044715


def _tc_body(src_ref, dst_ref, pop_ref, dmat_ref,
             w1t_ref, w2h_ref, b2_ref, lab_ref, out_ref):
    pop = pop_ref[...]
    inv = jnp.where(pop == 0.0, 0.0,
                    1.0 / jnp.where(pop == 0.0, 1.0, pop))
    a = src_ref[...] * inv
    b = dst_ref[...] * inv

    acc = jnp.zeros((P, V2), jnp.float32)
    for h in range(HID):
        u = w1t_ref[h:h + 1, 0:1]
        w = w1t_ref[h:h + 1, 1:2]
        x = a * u + b * w + dmat_ref[h:h + 1, :]
        s = x * x
        th = jnp.tanh(x * (s * _GKA + _GK))
        v = x * w2h_ref[h:h + 1, 0:1]
        acc = acc + v + v * th
    logits = acc + b2_ref[0:1, 0:1]

    col = lax.broadcasted_iota(jnp.int32, (1, V2), 1)
    valid = col < NUM_ITEMS
    neg = jnp.where(valid, logits, -jnp.inf)
    m = jnp.max(neg, axis=1, keepdims=True)
    s = jnp.sum(jnp.where(valid, jnp.exp(logits - m), 0.0),
                axis=1, keepdims=True)
    lab = lab_ref[0]
    pick = jnp.sum(jnp.where(col == lab, logits, 0.0),
                   axis=1, keepdims=True)
    loss = jnp.log(s) + m - pick
    out_ref[...] = loss.reshape(1, P, 1)


def _tc_loss(src_g, dst_g, pop2, popn, W1, b1, W2, b2, lab, n):
    nblk = n // P
    dmat = jnp.pad(W1[2][:, None] * popn + b1[:, None],
                   ((0, 0), (0, V2 - V)))
    return pl.pallas_call(
        _tc_body,
        grid=(nblk,),
        in_specs=[
            pl.BlockSpec((P, V2), lambda i: (i, 0)),
            pl.BlockSpec((P, V2), lambda i: (i, 0)),
            pl.BlockSpec((1, V2), lambda i: (0, 0)),
            pl.BlockSpec((HID, V2), lambda i: (0, 0)),
            pl.BlockSpec((HID, 3), lambda i: (0, 0)),
            pl.BlockSpec((HID, 1), lambda i: (0, 0)),
            pl.BlockSpec((1, 1), lambda i: (0, 0)),
            pl.BlockSpec((1, P, 1), lambda i: (i, 0, 0)),
        ],
        out_specs=pl.BlockSpec((1, P, 1), lambda i: (i, 0, 0)),
        out_shape=jax.ShapeDtypeStruct((nblk, P, 1), jnp.float32),
    )(src_g, dst_g, pop2, dmat,
      W1.T, (0.5 * W2), b2.reshape(1, 1),
      lab.reshape(nblk, P, 1)).reshape(n)


@jax.jit
def kernel(st_src, st_dst, pop_biases, pop_biases_norm, W1, b1, W2, b2,
           masked_sequences, labels, positions):
    B, L = masked_sequences.shape
    n = B * L

    seqs = jnp.maximum(masked_sequences, 0)
    pad_col = jnp.full((B, 1), PAD_ID, dtype=seqs.dtype)
    src_idx = jnp.concatenate([pad_col, seqs[:, :-1]], axis=1).reshape(n)
    dst_idx = jnp.concatenate([seqs[:, 1:], pad_col], axis=1).reshape(n)
    zpad = jnp.zeros((NPAD - n,), jnp.int32)
    src_idx = jnp.concatenate([src_idx, zpad])
    dst_idx = jnp.concatenate([dst_idx, zpad])
    lab = jnp.maximum(labels, 0).reshape(n)

    cpad = ((0, 0), (0, V2 - V))
    gather = _make_sc_gather(n)
    src2 = jnp.pad(st_src, cpad)
    src_g = gather(src2, src_idx)
    dst2 = jnp.pad(st_dst, cpad)
    dst_g = gather(dst2, dst_idx)
    pop2 = jnp.pad(pop_biases, cpad)

    return _tc_loss(src_g, dst_g, pop2, pop_biases_norm.reshape(1, V),
                    W1, b1, W2, b2, lab, n)


# speedup vs baseline: 1.4087x; 1.2802x over previous
import functools

import jax
import jax.numpy as jnp
from jax import lax
from jax.experimental import pallas as pl
from jax.experimental.pallas import tpu as pltpu
from jax.experimental.pallas import tpu_sc as plsc

V = 2000
V2 = 2048
NUM_ITEMS = V - 2
PAD_ID = NUM_ITEMS
HID = 32
P = 16
NPAD = 1024
RPAD = 40


def _pad_body(src_ref, dst_ref, osrc_ref, odst_ref):
    osrc_ref[:, 0:V] = src_ref[...]
    odst_ref[:, 0:V] = dst_ref[...]


def _pad_tables(st_src, st_dst):
    nblk = V // RPAD
    return pl.pallas_call(
        _pad_body,
        grid=(nblk,),
        in_specs=[
            pl.BlockSpec((RPAD, V), lambda i: (i, 0)),
            pl.BlockSpec((RPAD, V), lambda i: (i, 0)),
        ],
        out_specs=[
            pl.BlockSpec((RPAD, V2), lambda i: (i, 0)),
            pl.BlockSpec((RPAD, V2), lambda i: (i, 0)),
        ],
        out_shape=[jax.ShapeDtypeStruct((V, V2), jnp.float32),
                   jax.ShapeDtypeStruct((V, V2), jnp.float32)],
    )(st_src, st_dst)


def _make_sc_gather(n):
    info = plsc.get_sparse_core_info()
    nc, ns = info.num_cores, info.num_subcores
    nw = nc * ns
    b_per_w = NPAD // nw
    mesh = plsc.VectorSubcoreMesh(core_axis_name="c", subcore_axis_name="s")

    @functools.partial(
        pl.kernel, mesh=mesh,
        out_type=pltpu.HBM((NPAD, V2), jnp.float32),
        scratch_types=[
            pltpu.VMEM((b_per_w,), jnp.int32),
            pltpu.VMEM((b_per_w, V2), jnp.float32),
            pltpu.SemaphoreType.DMA,
        ],
    )
    def sc_gather(table_hbm, idx_hbm, out, idx_v, rows, sem):
        wid = lax.axis_index("s") * nc + lax.axis_index("c")
        base = wid * b_per_w
        nactive = -(-n // b_per_w)

        @pl.when(wid < nactive)
        def _():
            pltpu.sync_copy(idx_hbm.at[pl.ds(base, b_per_w)], idx_v)
            pltpu.async_copy(table_hbm.at[idx_v], rows, sem).wait()
            pltpu.sync_copy(rows, out.at[pl.ds(base, b_per_w)])

    return sc_gather


_GK = 0.7978845608028654
_GKA = _GK * 0.044715


def _tc_body(src_ref, dst_ref, pop_ref, dmat_ref,
             w1t_ref, w2h_ref, b2_ref, lab_ref, out_ref):
    pop = pop_ref[...]
    inv = jnp.where(pop == 0.0, 0.0,
                    1.0 / jnp.where(pop == 0.0, 1.0, pop))
    a = (src_ref[...] * inv).astype(jnp.bfloat16)
    b = (dst_ref[...] * inv).astype(jnp.bfloat16)
    dm = dmat_ref[...].astype(jnp.bfloat16)
    w1b = w1t_ref[...].astype(jnp.bfloat16)
    w2b = w2h_ref[...].astype(jnp.bfloat16)

    acc = jnp.zeros((P, V2), jnp.bfloat16)
    for h in range(HID):
        u = w1b[h:h + 1, 0:1]
        w = w1b[h:h + 1, 1:2]
        x = a * u + b * w + dm[h:h + 1, :]
        s = x * x
        th = jnp.tanh(x * (s * jnp.bfloat16(_GKA) + jnp.bfloat16(_GK)))
        v = x * w2b[h:h + 1, 0:1]
        acc = acc + v + v * th
    logits = acc.astype(jnp.float32) + b2_ref[0:1, 0:1]

    col = lax.broadcasted_iota(jnp.int32, (1, V2), 1)
    valid = col < NUM_ITEMS
    neg = jnp.where(valid, logits, -jnp.inf)
    m = jnp.max(neg, axis=1, keepdims=True)
    s = jnp.sum(jnp.where(valid, jnp.exp(logits - m), 0.0),
                axis=1, keepdims=True)
    lab = lab_ref[0]
    pick = jnp.sum(jnp.where(col == lab, logits, 0.0),
                   axis=1, keepdims=True)
    loss = jnp.log(s) + m - pick
    out_ref[...] = loss.reshape(1, P, 1)


def _tc_loss(src_g, dst_g, pop2, popn, W1, b1, W2, b2, lab, n):
    nblk = n // P
    dmat = jnp.pad(W1[2][:, None] * popn + b1[:, None],
                   ((0, 0), (0, V2 - V)))
    return pl.pallas_call(
        _tc_body,
        grid=(nblk,),
        in_specs=[
            pl.BlockSpec((P, V2), lambda i: (i, 0)),
            pl.BlockSpec((P, V2), lambda i: (i, 0)),
            pl.BlockSpec((1, V2), lambda i: (0, 0)),
            pl.BlockSpec((HID, V2), lambda i: (0, 0)),
            pl.BlockSpec((HID, 3), lambda i: (0, 0)),
            pl.BlockSpec((HID, 1), lambda i: (0, 0)),
            pl.BlockSpec((1, 1), lambda i: (0, 0)),
            pl.BlockSpec((1, P, 1), lambda i: (i, 0, 0)),
        ],
        out_specs=pl.BlockSpec((1, P, 1), lambda i: (i, 0, 0)),
        out_shape=jax.ShapeDtypeStruct((nblk, P, 1), jnp.float32),
    )(src_g, dst_g, pop2, dmat,
      W1.T, (0.5 * W2), b2.reshape(1, 1),
      lab.reshape(nblk, P, 1)).reshape(n)


@jax.jit
def kernel(st_src, st_dst, pop_biases, pop_biases_norm, W1, b1, W2, b2,
           masked_sequences, labels, positions):
    B, L = masked_sequences.shape
    n = B * L

    seqs = jnp.maximum(masked_sequences, 0)
    pad_col = jnp.full((B, 1), PAD_ID, dtype=seqs.dtype)
    src_idx = jnp.concatenate([pad_col, seqs[:, :-1]], axis=1).reshape(n)
    dst_idx = jnp.concatenate([seqs[:, 1:], pad_col], axis=1).reshape(n)
    zpad = jnp.zeros((NPAD - n,), jnp.int32)
    src_idx = jnp.concatenate([src_idx, zpad])
    dst_idx = jnp.concatenate([dst_idx, zpad])
    lab = jnp.maximum(labels, 0).reshape(n)

    cpad = ((0, 0), (0, V2 - V))
    gather = _make_sc_gather(n)
    src2 = jnp.pad(st_src, cpad)
    src_g = gather(src2, src_idx)
    dst2 = jnp.pad(st_dst, cpad)
    dst_g = gather(dst2, dst_idx)
    pop2 = jnp.pad(pop_biases, cpad)

    return _tc_loss(src_g, dst_g, pop2, pop_biases_norm.reshape(1, V),
                    W1, b1, W2, b2, lab, n)


# speedup vs baseline: 1.4647x; 1.0397x over previous
import functools

import jax
import jax.numpy as jnp
from jax import lax
from jax.experimental import pallas as pl
from jax.experimental.pallas import tpu as pltpu
from jax.experimental.pallas import tpu_sc as plsc

V = 2000
V2 = 2048
NUM_ITEMS = V - 2
PAD_ID = NUM_ITEMS
HID = 32
P = 32
NPAD = 1024
RPAD = 40


def _pad_body(src_ref, dst_ref, osrc_ref, odst_ref):
    osrc_ref[:, 0:V] = src_ref[...]
    odst_ref[:, 0:V] = dst_ref[...]


def _pad_tables(st_src, st_dst):
    nblk = V // RPAD
    return pl.pallas_call(
        _pad_body,
        grid=(nblk,),
        in_specs=[
            pl.BlockSpec((RPAD, V), lambda i: (i, 0)),
            pl.BlockSpec((RPAD, V), lambda i: (i, 0)),
        ],
        out_specs=[
            pl.BlockSpec((RPAD, V2), lambda i: (i, 0)),
            pl.BlockSpec((RPAD, V2), lambda i: (i, 0)),
        ],
        out_shape=[jax.ShapeDtypeStruct((V, V2), jnp.float32),
                   jax.ShapeDtypeStruct((V, V2), jnp.float32)],
    )(st_src, st_dst)


def _make_sc_gather(n):
    info = plsc.get_sparse_core_info()
    nc, ns = info.num_cores, info.num_subcores
    nw = nc * ns
    b_per_w = NPAD // nw
    mesh = plsc.VectorSubcoreMesh(core_axis_name="c", subcore_axis_name="s")

    @functools.partial(
        pl.kernel, mesh=mesh,
        out_type=pltpu.HBM((NPAD, V2), jnp.float32),
        scratch_types=[
            pltpu.VMEM((b_per_w,), jnp.int32),
            pltpu.VMEM((b_per_w, V2), jnp.float32),
            pltpu.SemaphoreType.DMA,
        ],
    )
    def sc_gather(table_hbm, idx_hbm, out, idx_v, rows, sem):
        wid = lax.axis_index("s") * nc + lax.axis_index("c")
        base = wid * b_per_w
        nactive = -(-n // b_per_w)

        @pl.when(wid < nactive)
        def _():
            pltpu.sync_copy(idx_hbm.at[pl.ds(base, b_per_w)], idx_v)
            pltpu.async_copy(table_hbm.at[idx_v], rows, sem).wait()
            pltpu.sync_copy(rows, out.at[pl.ds(base, b_per_w)])

    return sc_gather


_GK = 0.7978845608028654
_GKA = _GK * 0.044715


def _tc_body(src_ref, dst_ref, pop_ref, dmat_ref,
             w1t_ref, w2h_ref, b2_ref, lab_ref, out_ref):
    pop = pop_ref[...]
    inv = jnp.where(pop == 0.0, 0.0,
                    1.0 / jnp.where(pop == 0.0, 1.0, pop))
    a = (src_ref[...] * inv).astype(jnp.bfloat16)
    b = (dst_ref[...] * inv).astype(jnp.bfloat16)
    dm = dmat_ref[...].astype(jnp.bfloat16)
    w1b = w1t_ref[...].astype(jnp.bfloat16)
    w2b = w2h_ref[...].astype(jnp.bfloat16)

    acc = jnp.zeros((P, V2), jnp.bfloat16)
    for h in range(HID):
        u = w1b[h:h + 1, 0:1]
        w = w1b[h:h + 1, 1:2]
        x = a * u + b * w + dm[h:h + 1, :]
        s = x * x
        th = jnp.tanh(x * (s * jnp.bfloat16(_GKA) + jnp.bfloat16(_GK)))
        v = x * w2b[h:h + 1, 0:1]
        acc = acc + v + v * th
    logits = acc.astype(jnp.float32) + b2_ref[0:1, 0:1]

    col = lax.broadcasted_iota(jnp.int32, (1, V2), 1)
    valid = col < NUM_ITEMS
    neg = jnp.where(valid, logits, -jnp.inf)
    m = jnp.max(neg, axis=1, keepdims=True)
    s = jnp.sum(jnp.where(valid, jnp.exp(logits - m), 0.0),
                axis=1, keepdims=True)
    lab = lab_ref[0]
    pick = jnp.sum(jnp.where(col == lab, logits, 0.0),
                   axis=1, keepdims=True)
    loss = jnp.log(s) + m - pick
    out_ref[...] = loss.reshape(1, P, 1)


def _tc_loss(src_g, dst_g, pop2, popn, W1, b1, W2, b2, lab, n):
    nblk = n // P
    dmat = jnp.pad(W1[2][:, None] * popn + b1[:, None],
                   ((0, 0), (0, V2 - V)))
    return pl.pallas_call(
        _tc_body,
        grid=(nblk,),
        in_specs=[
            pl.BlockSpec((P, V2), lambda i: (i, 0)),
            pl.BlockSpec((P, V2), lambda i: (i, 0)),
            pl.BlockSpec((1, V2), lambda i: (0, 0)),
            pl.BlockSpec((HID, V2), lambda i: (0, 0)),
            pl.BlockSpec((HID, 3), lambda i: (0, 0)),
            pl.BlockSpec((HID, 1), lambda i: (0, 0)),
            pl.BlockSpec((1, 1), lambda i: (0, 0)),
            pl.BlockSpec((1, P, 1), lambda i: (i, 0, 0)),
        ],
        out_specs=pl.BlockSpec((1, P, 1), lambda i: (i, 0, 0)),
        out_shape=jax.ShapeDtypeStruct((nblk, P, 1), jnp.float32),
    )(src_g, dst_g, pop2, dmat,
      W1.T, (0.5 * W2), b2.reshape(1, 1),
      lab.reshape(nblk, P, 1)).reshape(n)


@jax.jit
def kernel(st_src, st_dst, pop_biases, pop_biases_norm, W1, b1, W2, b2,
           masked_sequences, labels, positions):
    B, L = masked_sequences.shape
    n = B * L

    seqs = jnp.maximum(masked_sequences, 0)
    pad_col = jnp.full((B, 1), PAD_ID, dtype=seqs.dtype)
    src_idx = jnp.concatenate([pad_col, seqs[:, :-1]], axis=1).reshape(n)
    dst_idx = jnp.concatenate([seqs[:, 1:], pad_col], axis=1).reshape(n)
    zpad = jnp.zeros((NPAD - n,), jnp.int32)
    src_idx = jnp.concatenate([src_idx, zpad])
    dst_idx = jnp.concatenate([dst_idx, zpad])
    lab = jnp.maximum(labels, 0).reshape(n)

    cpad = ((0, 0), (0, V2 - V))
    gather = _make_sc_gather(n)
    src2 = jnp.pad(st_src, cpad)
    src_g = gather(src2, src_idx)
    dst2 = jnp.pad(st_dst, cpad)
    dst_g = gather(dst2, dst_idx)
    pop2 = jnp.pad(pop_biases, cpad)

    return _tc_loss(src_g, dst_g, pop2, pop_biases_norm.reshape(1, V),
                    W1, b1, W2, b2, lab, n)
